# Initial kernel scaffold; baseline (speedup 1.0000x reference)
import jax
import jax.numpy as jnp
from jax.experimental import pallas as pl


def kernel(boxes, scores):
    raise NotImplementedError("write your pallas kernel here")


# speedup vs baseline: 1.2880x; 1.2880x over previous
import functools
import numpy as np
import jax
import jax.numpy as jnp
from jax.experimental import pallas as pl

_IMG_H = 640.0
_IMG_W = 640.0
_STRIDES = (8, 16, 32)
_SCORE_THR = 0.001
_IOU_THR = 0.65
_MAX_DET = 200
_NUM_CANDS = 2048
_NUM_CLASSES = 80


def _anchors_np():
    grids = []
    strl = []
    for stride in _STRIDES:
        h = int(_IMG_H) // stride
        w = int(_IMG_W) // stride
        yv, xv = np.meshgrid(np.arange(h), np.arange(w), indexing='ij')
        grid = np.stack((xv, yv), 2).reshape(-1, 2)
        grids.append(grid)
        strl.append(np.full((grid.shape[0], 1), stride))
    s = np.concatenate(strl, 0).astype(np.float32)
    off = s * np.concatenate(grids, 0).astype(np.float32)
    xc = off[:, 0:1]
    yc = off[:, 1:2]
    return np.concatenate(
        [(2 * yc - s) / 2, (2 * xc - s) / 2, (2 * yc + s) / 2, (2 * xc + s) / 2], -1)


def _nms_kernel(ty, tx, th, tw, ay1, ax1, ay2, ax2, lab_f, sc,
                oy1, ox1, oy2, ox2, osc, olab):
    B, K = sc.shape

    a_h = ay2[...] - ay1[...]
    a_w = ax2[...] - ax1[...]
    a_yc = ay1[...] + 0.5 * a_h
    a_xc = ax1[...] + 0.5 * a_w
    yc = ty[...] * a_h + a_yc
    xc = tx[...] * a_w + a_xc
    hh = jnp.exp(th[...]) * a_h
    ww = jnp.exp(tw[...]) * a_w
    y1 = jnp.clip(yc - 0.5 * hh, 0.0, _IMG_H)
    x1 = jnp.clip(xc - 0.5 * ww, 0.0, _IMG_W)
    y2 = jnp.clip(yc + 0.5 * hh, 0.0, _IMG_H)
    x2 = jnp.clip(xc + 0.5 * ww, 0.0, _IMG_W)

    lab = lab_f[...]
    off = lab * 1e4
    Y1 = y1 + off
    X1 = x1 + off
    Y2 = y2 + off
    X2 = x2 + off
    areas = (Y2 - Y1) * (X2 - X1)

    iota_k = jax.lax.broadcasted_iota(jnp.int32, (B, K), 1)
    iota_o = jax.lax.broadcasted_iota(jnp.int32, (1, 256), 1)

    def body(i, state):
        rem, ay1a, ax1a, ay2a, ax2a, asca, alaba = state
        v = jnp.max(rem, axis=1, keepdims=True)
        pos = jnp.where(rem == v, iota_k, K)
        jmin = jnp.min(pos, axis=1, keepdims=True)
        onehot = (iota_k == jmin).astype(jnp.float32)

        s_y1 = jnp.sum(y1 * onehot, axis=1, keepdims=True)
        s_x1 = jnp.sum(x1 * onehot, axis=1, keepdims=True)
        s_y2 = jnp.sum(y2 * onehot, axis=1, keepdims=True)
        s_x2 = jnp.sum(x2 * onehot, axis=1, keepdims=True)
        s_lab = jnp.sum(lab * onehot, axis=1, keepdims=True)

        s_off = s_lab * 1e4
        SY1 = s_y1 + s_off
        SX1 = s_x1 + s_off
        SY2 = s_y2 + s_off
        SX2 = s_x2 + s_off
        s_area = (SY2 - SY1) * (SX2 - SX1)

        iy1 = jnp.maximum(SY1, Y1)
        ix1 = jnp.maximum(SX1, X1)
        iy2 = jnp.minimum(SY2, Y2)
        ix2 = jnp.minimum(SX2, X2)
        inter = jnp.clip(iy2 - iy1, 0.0) * jnp.clip(ix2 - ix1, 0.0)
        union = s_area + areas - inter
        iou = inter / jnp.maximum(union, 1e-9)
        rem = jnp.where(jnp.logical_or(iou > _IOU_THR, iota_k == jmin), -1.0, rem)

        valid = (v > 0.0).astype(jnp.float32)
        lane = (iota_o == i).astype(jnp.float32)
        ay1a = ay1a + (valid * s_y1) * lane
        ax1a = ax1a + (valid * s_x1) * lane
        ay2a = ay2a + (valid * s_y2) * lane
        ax2a = ax2a + (valid * s_x2) * lane
        asca = asca + (valid * v) * lane
        alaba = alaba + (valid * s_lab) * lane
        return rem, ay1a, ax1a, ay2a, ax2a, asca, alaba

    z = jnp.zeros((B, 256), jnp.float32)
    state = (sc[...], z, z, z, z, z, z)
    state = jax.lax.fori_loop(0, _MAX_DET, body, state)
    _, ay1a, ax1a, ay2a, ax2a, asca, alaba = state
    oy1[...] = ay1a
    ox1[...] = ax1a
    oy2[...] = ay2a
    ox2[...] = ax2a
    osc[...] = asca
    olab[...] = alaba


def kernel(boxes, scores):
    B = boxes.shape[0]
    anchors = jnp.asarray(_anchors_np())

    flat = scores.reshape(B, -1)
    flat = jnp.where(flat >= _SCORE_THR, flat, -1.0)
    top_scores, top_idx = jax.lax.top_k(flat, _NUM_CANDS)
    box_idx = top_idx // _NUM_CLASSES
    labels = top_idx % _NUM_CLASSES

    rel = jnp.take_along_axis(boxes, box_idx[..., None], axis=1)
    anc = jnp.take_along_axis(anchors[None], box_idx[..., None], axis=1)

    args = (
        rel[..., 0], rel[..., 1], rel[..., 2], rel[..., 3],
        anc[..., 0], anc[..., 1], anc[..., 2], anc[..., 3],
        labels.astype(jnp.float32), top_scores,
    )
    outs = pl.pallas_call(
        _nms_kernel,
        out_shape=[jax.ShapeDtypeStruct((B, 256), jnp.float32)] * 6,
    )(*args)
    oy1, ox1, oy2, ox2, osc, olab = outs
    out_boxes = jnp.stack(
        [oy1[:, :_MAX_DET], ox1[:, :_MAX_DET], oy2[:, :_MAX_DET], ox2[:, :_MAX_DET]],
        axis=-1)
    out_scores = osc[:, :_MAX_DET]
    out_labels = olab[:, :_MAX_DET].astype(jnp.int32)
    return out_boxes, out_scores, out_labels
